# baseline (speedup 1.0000x reference)
import functools

import jax
import jax.numpy as jnp
from jax import lax
from jax.experimental import pallas as pl
from jax.experimental.pallas import tpu as pltpu
from jax.experimental.pallas import tpu_sc as plsc

R, A, M, V, D = 64, 26, 200, 100000, 64
L = 16
NC, NS = 2, 16
NW = NC * NS
SEG = R * A
PER_W = SEG // NW
N_D = D // L
GCHUNKS = ((0, 128), (128, 72))
N_FULL = M // L
TAIL_OFF = M - L


_GDN = lax.GatherDimensionNumbers(
    offset_dims=(), collapsed_slice_dims=(0,), start_index_map=(0,))


def _splat(vec, j):
  idx = jnp.full((L, 1), j, dtype=jnp.int32)
  return lax.gather(vec, idx, _GDN, (1,),
                    mode=lax.GatherScatterMode.PROMISE_IN_BOUNDS)


def _sc_partials(table, idx, slack, emb_aux):
  mesh = plsc.VectorSubcoreMesh(core_axis_name="c", subcore_axis_name="s")

  @functools.partial(
      pl.kernel,
      out_type=jax.ShapeDtypeStruct((NW, L), jnp.float32),
      mesh=mesh,
      compiler_params=pltpu.CompilerParams(use_tc_tiling_on_sc=False,
                                           needs_layout_passes=False),
      scratch_types=[
          pltpu.VMEM((PER_W, M), jnp.int32),
          pltpu.VMEM((PER_W, M), jnp.float32),
          pltpu.VMEM((PER_W, D), jnp.float32),
          pltpu.VMEM((4, M, D), jnp.float32),
          pltpu.VMEM((L,), jnp.float32),
          pltpu.SemaphoreType.DMA,
          pltpu.SemaphoreType.DMA,
          pltpu.SemaphoreType.DMA,
          pltpu.SemaphoreType.DMA,
      ],
  )
  def k(table_h, idx_h, slack_h, emb_h, out_h,
        idx_v, slack_v, emb_v, rows_v, stage_v, sem0, sem1, sem2, sem3):
    sems = (sem0, sem1, sem2, sem3)
    wid = lax.axis_index("s") * NC + lax.axis_index("c")
    base = wid * PER_W

    pltpu.sync_copy(idx_h.at[pl.ds(base, PER_W), :], idx_v)
    pltpu.sync_copy(slack_h.at[pl.ds(base, PER_W), :], slack_v)
    pltpu.sync_copy(emb_h.at[pl.ds(base, PER_W), :], emb_v)

    def fire(q, buf):
      for off, n in GCHUNKS:
        pltpu.async_copy(
            table_h.at[idx_v.at[q, pl.ds(off, n)]],
            rows_v.at[buf, pl.ds(off, n), :],
            sems[buf])

    def drain(q, buf):
      for off, n in GCHUNKS:
        pltpu.make_async_copy(
            table_h.at[idx_v.at[q, pl.ds(off, n)]],
            rows_v.at[buf, pl.ds(off, n), :],
            sems[buf]).wait()

    fire(0, 0)
    fire(1, 1)
    fire(2, 2)
    fire(3, 3)

    lane = lax.broadcasted_iota(jnp.int32, (L,), 0)
    tail_mask = (lane >= (L - (M - N_FULL * L))).astype(jnp.float32)
    zeros = jnp.zeros((L,), jnp.float32)

    def do_pair(q, buf, carry):
      rmax, wmin, slsq, term_acc = carry
      drain(q, buf)
      acc = [zeros] * N_D
      for ci in range(N_FULL + 1):
        if ci < N_FULL:
          off, j0 = ci * L, 0
        else:
          off, j0 = TAIL_OFF, L - (M - N_FULL * L)
        s_vec = slack_v[q, pl.ds(off, L)]
        for j in range(j0, L):
          m = off + j
          w = _splat(s_vec, j)
          for kk in range(N_D):
            acc[kk] = acc[kk] + w * rows_v[buf, m, pl.ds(kk * L, L)]
      fire(jnp.minimum(q + 4, PER_W - 1), buf)
      ssq_v = zeros
      for kk in range(N_D):
        d_v = emb_v[q, pl.ds(kk * L, L)] - acc[kk]
        ssq_v = ssq_v + d_v * d_v
      ssq = jnp.sum(ssq_v)
      rmax = jnp.maximum(rmax, ssq)
      ssum_v = zeros
      for ci in range(N_FULL + 1):
        off = ci * L if ci < N_FULL else TAIL_OFF
        s = slack_v[q, pl.ds(off, L)]
        if ci == N_FULL:
          s = s * tail_mask
        term_acc = term_acc + (jnp.maximum(1.0, s) - 1.0 - jnp.minimum(0.0, s))
        ssum_v = ssum_v + s
      ssum = jnp.sum(ssum_v)
      slsq = slsq + (ssum - 1.0) * (ssum - 1.0)
      is_end = (q % A) == (A - 1)
      wmin = jnp.where(is_end, jnp.minimum(wmin, rmax), wmin)
      rmax = jnp.where(is_end, jnp.float32(0.0), rmax)
      return rmax, wmin, slsq, term_acc

    def body(i, carry):
      q0 = i * 4
      carry = do_pair(q0, 0, carry)
      carry = do_pair(q0 + 1, 1, carry)
      carry = do_pair(q0 + 2, 2, carry)
      carry = do_pair(q0 + 3, 3, carry)
      return carry

    init = (jnp.float32(0.0), jnp.float32(jnp.inf), jnp.float32(0.0), zeros)
    rmax, wmin, slsq, term_acc = lax.fori_loop(0, PER_W // 4, body, init)

    drain(PER_W - 1, 0)
    drain(PER_W - 1, 1)
    drain(PER_W - 1, 2)
    drain(PER_W - 1, 3)

    slack_part = slsq + jnp.sum(term_acc)
    out_vec = jnp.where(lane == 0, wmin,
                        jnp.where(lane == 1, slack_part, jnp.float32(0.0)))
    stage_v[...] = out_vec
    pltpu.sync_copy(stage_v, out_h.at[wid])

  return k(table, idx, slack, emb_aux)


def _tc_finish(partials):
  def body(p_ref, l1_ref, l2_ref):
    x = p_ref[...]
    l1_ref[...] = jnp.sqrt(jnp.min(x[:, 0]) / jnp.float32(D)).reshape(1, 1)
    l2_ref[...] = jnp.sum(x[:, 1]).reshape(1, 1)

  return pl.pallas_call(
      body,
      out_shape=(jax.ShapeDtypeStruct((1, 1), jnp.float32),
                 jax.ShapeDtypeStruct((1, 1), jnp.float32)),
  )(partials)


@jax.jit
def kernel(table, emb_aux, slack, source_idx, Temp):
  idx = source_idx.reshape(SEG, M).astype(jnp.int32)
  slack2 = slack.reshape(SEG, M)
  emb2 = emb_aux.reshape(SEG, D)
  partials = _sc_partials(table, idx, slack2, emb2)
  l1, l2 = _tc_finish(partials)
  return l1[0, 0], l2[0, 0]


# speedup vs baseline: 1.0424x; 1.0424x over previous
import functools

import jax
import jax.numpy as jnp
from jax import lax
from jax.experimental import pallas as pl
from jax.experimental.pallas import tpu as pltpu
from jax.experimental.pallas import tpu_sc as plsc

R, A, M, V, D = 64, 26, 200, 100000, 64
L = 16
NC, NS = 2, 16
NW = NC * NS
SEG = R * A
PER_W = SEG // NW
N_D = D // L
GCHUNKS = ((0, 128), (128, 72))
N_FULL = M // L
TAIL_OFF = M - L


_GDN = lax.GatherDimensionNumbers(
    offset_dims=(), collapsed_slice_dims=(0,), start_index_map=(0,))


def _splat(vec, j):
  idx = jnp.full((L, 1), j, dtype=jnp.int32)
  return lax.gather(vec, idx, _GDN, (1,),
                    mode=lax.GatherScatterMode.PROMISE_IN_BOUNDS)


def _sc_partials(table, idx, slack, emb_aux):
  mesh = plsc.VectorSubcoreMesh(core_axis_name="c", subcore_axis_name="s")

  @functools.partial(
      pl.kernel,
      out_type=jax.ShapeDtypeStruct((NW, L), jnp.float32),
      mesh=mesh,
      compiler_params=pltpu.CompilerParams(use_tc_tiling_on_sc=False,
                                           needs_layout_passes=False),
      scratch_types=[
          pltpu.VMEM((PER_W, M), jnp.int32),
          pltpu.VMEM((PER_W, M), jnp.float32),
          pltpu.VMEM((PER_W, D), jnp.float32),
          pltpu.VMEM((2, M, D), jnp.float32),
          pltpu.VMEM((L,), jnp.float32),
          pltpu.SemaphoreType.DMA,
          pltpu.SemaphoreType.DMA,
      ],
  )
  def k(table_h, idx_h, slack_h, emb_h, out_h,
        idx_v, slack_v, emb_v, rows_v, stage_v, sem0, sem1):
    sems = (sem0, sem1)
    wid = lax.axis_index("s") * NC + lax.axis_index("c")
    base = wid * PER_W

    pltpu.sync_copy(idx_h.at[pl.ds(base, PER_W), :], idx_v)
    pltpu.sync_copy(slack_h.at[pl.ds(base, PER_W), :], slack_v)
    pltpu.sync_copy(emb_h.at[pl.ds(base, PER_W), :], emb_v)

    def fire(q, buf):
      for off, n in GCHUNKS:
        pltpu.async_copy(
            table_h.at[idx_v.at[q, pl.ds(off, n)]],
            rows_v.at[buf, pl.ds(off, n), :],
            sems[buf])

    def drain(q, buf):
      for off, n in GCHUNKS:
        pltpu.make_async_copy(
            table_h.at[idx_v.at[q, pl.ds(off, n)]],
            rows_v.at[buf, pl.ds(off, n), :],
            sems[buf]).wait()

    fire(0, 0)
    fire(1, 1)

    lane = lax.broadcasted_iota(jnp.int32, (L,), 0)
    tail_mask = (lane >= (L - (M - N_FULL * L))).astype(jnp.float32)
    zeros = jnp.zeros((L,), jnp.float32)

    def do_pair(q, buf, carry):
      rmax, wmin, slsq, term_acc = carry
      drain(q, buf)
      acc = [zeros] * (2 * N_D)
      for ci in range(N_FULL + 1):
        if ci < N_FULL:
          off, j0 = ci * L, 0
        else:
          off, j0 = TAIL_OFF, L - (M - N_FULL * L)
        s_vec = slack_v[q, pl.ds(off, L)]
        for b0 in range(j0, L, 8):
          ws = [_splat(s_vec, j) for j in range(b0, b0 + 8)]
          for jj, w in enumerate(ws):
            m = off + b0 + jj
            p = jj & 1
            for kk in range(N_D):
              acc[2 * kk + p] = acc[2 * kk + p] + w * rows_v[buf, m, pl.ds(kk * L, L)]
      acc = [acc[2 * kk] + acc[2 * kk + 1] for kk in range(N_D)]
      fire(jnp.minimum(q + 2, PER_W - 1), buf)
      ssq_v = zeros
      for kk in range(N_D):
        d_v = emb_v[q, pl.ds(kk * L, L)] - acc[kk]
        ssq_v = ssq_v + d_v * d_v
      ssq = jnp.sum(ssq_v)
      rmax = jnp.maximum(rmax, ssq)
      ssum_v = zeros
      for ci in range(N_FULL + 1):
        off = ci * L if ci < N_FULL else TAIL_OFF
        s = slack_v[q, pl.ds(off, L)]
        if ci == N_FULL:
          s = s * tail_mask
        term_acc = term_acc + (jnp.maximum(1.0, s) - 1.0 - jnp.minimum(0.0, s))
        ssum_v = ssum_v + s
      ssum = jnp.sum(ssum_v)
      slsq = slsq + (ssum - 1.0) * (ssum - 1.0)
      is_end = (q % A) == (A - 1)
      wmin = jnp.where(is_end, jnp.minimum(wmin, rmax), wmin)
      rmax = jnp.where(is_end, jnp.float32(0.0), rmax)
      return rmax, wmin, slsq, term_acc

    def body(i, carry):
      q0 = i * 2
      carry = do_pair(q0, 0, carry)
      carry = do_pair(q0 + 1, 1, carry)
      return carry

    init = (jnp.float32(0.0), jnp.float32(jnp.inf), jnp.float32(0.0), zeros)
    rmax, wmin, slsq, term_acc = lax.fori_loop(0, PER_W // 2, body, init)

    drain(PER_W - 1, 0)
    drain(PER_W - 1, 1)

    slack_part = slsq + jnp.sum(term_acc)
    out_vec = jnp.where(lane == 0, wmin,
                        jnp.where(lane == 1, slack_part, jnp.float32(0.0)))
    stage_v[...] = out_vec
    pltpu.sync_copy(stage_v, out_h.at[wid])

  return k(table, idx, slack, emb_aux)


def _tc_finish(partials):
  def body(p_ref, l1_ref, l2_ref):
    x = p_ref[...]
    l1_ref[...] = jnp.sqrt(jnp.min(x[:, 0]) / jnp.float32(D)).reshape(1, 1)
    l2_ref[...] = jnp.sum(x[:, 1]).reshape(1, 1)

  return pl.pallas_call(
      body,
      out_shape=(jax.ShapeDtypeStruct((1, 1), jnp.float32),
                 jax.ShapeDtypeStruct((1, 1), jnp.float32)),
  )(partials)


@jax.jit
def kernel(table, emb_aux, slack, source_idx, Temp):
  idx = source_idx.reshape(SEG, M).astype(jnp.int32)
  slack2 = slack.reshape(SEG, M)
  emb2 = emb_aux.reshape(SEG, D)
  partials = _sc_partials(table, idx, slack2, emb2)
  l1, l2 = _tc_finish(partials)
  return l1[0, 0], l2[0, 0]


# speedup vs baseline: 1.0789x; 1.0351x over previous
import functools

import jax
import jax.numpy as jnp
from jax import lax
from jax.experimental import pallas as pl
from jax.experimental.pallas import tpu as pltpu
from jax.experimental.pallas import tpu_sc as plsc

R, A, M, V, D = 64, 26, 200, 100000, 64
L = 16
NC, NS = 2, 16
NW = NC * NS
SEG = R * A
PER_W = SEG // NW
N_D = D // L
GCHUNKS = ((0, 128), (128, 72))
N_FULL = M // L
TAIL_OFF = M - L


_GDN = lax.GatherDimensionNumbers(
    offset_dims=(), collapsed_slice_dims=(0,), start_index_map=(0,))


def _splat(vec, j):
  idx = jnp.full((L, 1), j, dtype=jnp.int32)
  return lax.gather(vec, idx, _GDN, (1,),
                    mode=lax.GatherScatterMode.PROMISE_IN_BOUNDS)


def _sc_partials(table, idx, slack, emb_aux):
  mesh = plsc.VectorSubcoreMesh(core_axis_name="c", subcore_axis_name="s")

  @functools.partial(
      pl.kernel,
      out_type=jax.ShapeDtypeStruct((NW, L), jnp.float32),
      mesh=mesh,
      compiler_params=pltpu.CompilerParams(use_tc_tiling_on_sc=False,
                                           needs_layout_passes=False),
      scratch_types=[
          pltpu.VMEM((PER_W, M), jnp.int32),
          pltpu.VMEM((PER_W, M), jnp.float32),
          pltpu.VMEM((PER_W, D), jnp.float32),
          pltpu.VMEM((2, M, D), jnp.float32),
          pltpu.VMEM((L,), jnp.float32),
          pltpu.SemaphoreType.DMA,
          pltpu.SemaphoreType.DMA,
      ],
  )
  def k(table_h, idx_h, slack_h, emb_h, out_h,
        idx_v, slack_v, emb_v, rows_v, stage_v, sem0, sem1):
    sems = (sem0, sem1)
    wid = lax.axis_index("s") * NC + lax.axis_index("c")
    base = wid * PER_W

    pltpu.sync_copy(idx_h.at[pl.ds(base, PER_W), :], idx_v)
    pltpu.sync_copy(slack_h.at[pl.ds(base, PER_W), :], slack_v)
    pltpu.sync_copy(emb_h.at[pl.ds(base, PER_W), :], emb_v)

    def fire(q, buf):
      for off, n in GCHUNKS:
        pltpu.async_copy(
            table_h.at[idx_v.at[q, pl.ds(off, n)]],
            rows_v.at[buf, pl.ds(off, n), :],
            sems[buf])

    def drain(q, buf):
      for off, n in GCHUNKS:
        pltpu.make_async_copy(
            table_h.at[idx_v.at[q, pl.ds(off, n)]],
            rows_v.at[buf, pl.ds(off, n), :],
            sems[buf]).wait()

    fire(0, 0)
    fire(1, 1)

    lane = lax.broadcasted_iota(jnp.int32, (L,), 0)
    tail_mask = (lane >= (L - (M - N_FULL * L))).astype(jnp.float32)
    zeros = jnp.zeros((L,), jnp.float32)

    def do_pair(q, buf, carry):
      rmax, wmin, slsq, term_acc = carry
      drain(q, buf)
      acc = [zeros] * N_D
      for ci in range(N_FULL + 1):
        if ci < N_FULL:
          off, j0 = ci * L, 0
        else:
          off, j0 = TAIL_OFF, L - (M - N_FULL * L)
        s_vec = slack_v[q, pl.ds(off, L)]
        for j in range(j0, L):
          m = off + j
          w = _splat(s_vec, j)
          for kk in range(N_D):
            acc[kk] = acc[kk] + w * rows_v[buf, m, pl.ds(kk * L, L)]
      fire(jnp.minimum(q + 2, PER_W - 1), buf)
      ssq_v = zeros
      for kk in range(N_D):
        d_v = emb_v[q, pl.ds(kk * L, L)] - acc[kk]
        ssq_v = ssq_v + d_v * d_v
      ssq = jnp.sum(ssq_v)
      rmax = jnp.maximum(rmax, ssq)
      ssum_v = zeros
      for ci in range(N_FULL + 1):
        off = ci * L if ci < N_FULL else TAIL_OFF
        s = slack_v[q, pl.ds(off, L)]
        if ci == N_FULL:
          s = s * tail_mask
        term_acc = term_acc + (jnp.maximum(1.0, s) - 1.0 - jnp.minimum(0.0, s))
        ssum_v = ssum_v + s
      ssum = jnp.sum(ssum_v)
      slsq = slsq + (ssum - 1.0) * (ssum - 1.0)
      is_end = (q % A) == (A - 1)
      wmin = jnp.where(is_end, jnp.minimum(wmin, rmax), wmin)
      rmax = jnp.where(is_end, jnp.float32(0.0), rmax)
      return rmax, wmin, slsq, term_acc

    def body(i, carry):
      q0 = i * 2
      carry = do_pair(q0, 0, carry)
      carry = do_pair(q0 + 1, 1, carry)
      return carry

    init = (jnp.float32(0.0), jnp.float32(jnp.inf), jnp.float32(0.0), zeros)
    rmax, wmin, slsq, term_acc = lax.fori_loop(0, PER_W // 2, body, init)

    drain(PER_W - 1, 0)
    drain(PER_W - 1, 1)

    slack_part = slsq + jnp.sum(term_acc)
    out_vec = jnp.where(lane == 0, wmin,
                        jnp.where(lane == 1, slack_part, jnp.float32(0.0)))
    stage_v[...] = out_vec
    pltpu.sync_copy(stage_v, out_h.at[wid])

  return k(table, idx, slack, emb_aux)


def _tc_finish(partials):
  def body(p_ref, l1_ref, l2_ref):
    x = p_ref[...]
    l1_ref[...] = jnp.sqrt(jnp.min(x[:, 0]) / jnp.float32(D)).reshape(1, 1)
    l2_ref[...] = jnp.sum(x[:, 1]).reshape(1, 1)

  return pl.pallas_call(
      body,
      out_shape=(jax.ShapeDtypeStruct((1, 1), jnp.float32),
                 jax.ShapeDtypeStruct((1, 1), jnp.float32)),
  )(partials)


@jax.jit
def kernel(table, emb_aux, slack, source_idx, Temp):
  idx = source_idx.reshape(SEG, M).astype(jnp.int32)
  slack2 = slack.reshape(SEG, M)
  emb2 = emb_aux.reshape(SEG, D)
  partials = _sc_partials(table, idx, slack2, emb2)
  l1, l2 = _tc_finish(partials)
  return l1[0, 0], l2[0, 0]
